# baseline (speedup 1.0000x reference)
import functools

import jax
import jax.numpy as jnp
from jax import lax
from jax.experimental import pallas as pl
from jax.experimental.pallas import tpu as pltpu
from jax.experimental.pallas import tpu_sc as plsc

_N = 10000
_NPAD = 10240
_E = 320000
_H = 128
_B = 4096
_NC = 2
_NS = 16
_CH = 128
_NCHT = 160
_EPT = _NCHT * _CH
_EP = _NS * _EPT
_RPT = _NPAD // _NS

_f32 = jnp.float32


def _mesh():
    return plsc.VectorSubcoreMesh(core_axis_name="c", subcore_axis_name="s")


_CHS = 80
_NCHS = _EPT // _CHS


def _edge_sweep(tab, src, dst, base, src_all, dch0, dch1,
                acc, rows0, rows1, semg0, semg1, semd0, semd1, scat_x):
    def fire(i, dch, rows, semd, semg):
        pltpu.async_copy(dst.at[pl.ds(base + i * _CHS, _CHS)], dch, semd)
        pltpu.async_copy(tab.at[src_all.at[pl.ds(i * _CHS, _CHS)]],
                         rows, semg)

    def drain(dch, rows, semd, semg):
        pltpu.make_async_copy(dst.at[pl.ds(base, _CHS)], dch, semd).wait()
        pltpu.make_async_copy(tab.at[src_all.at[pl.ds(0, _CHS)]],
                              rows, semg).wait()

    def scat(dch, rows):
        pltpu.sync_copy(rows, acc.at[dch], add=True)
        scat_x(dch)

    pltpu.sync_copy(src.at[pl.ds(base, _EPT)], src_all)
    fire(0, dch0, rows0, semd0, semg0)

    def step(k, carry):
        i0 = k * 2
        fire(i0 + 1, dch1, rows1, semd1, semg1)
        drain(dch0, rows0, semd0, semg0)
        scat(dch0, rows0)
        fire(i0 + 2, dch0, rows0, semd0, semg0)
        drain(dch1, rows1, semd1, semg1)
        scat(dch1, rows1)
        return carry

    lax.fori_loop(0, _NCHS // 2 - 1, step, 0)
    fire(_NCHS - 1, dch1, rows1, semd1, semg1)
    drain(dch0, rows0, semd0, semg0)
    scat(dch0, rows0)
    drain(dch1, rows1, semd1, semg1)
    scat(dch1, rows1)


def _sweep0_body(tab, src, dst, zrows, zc, ones_h, out_sum, out_cnt,
                 acc, cacc, rows0, rows1, dch0, dch1, src_all, ones_v,
                 semg0, semg1, semd0, semd1):
    c = lax.axis_index("c")
    s = lax.axis_index("s")
    pltpu.sync_copy(zrows, acc.at[pl.ds(s * _RPT, _RPT)])
    pltpu.sync_copy(zc, cacc.at[pl.ds(s * _RPT, _RPT)])
    pltpu.sync_copy(ones_h, ones_v)
    base = c * _EP + s * _EPT
    plsc.subcore_barrier()

    def scat_cnt(dch):
        pltpu.sync_copy(ones_v, cacc.at[dch], add=True)

    _edge_sweep(tab, src, dst, base, src_all, dch0, dch1,
                acc, rows0, rows1, semg0, semg1, semd0, semd1, scat_cnt)
    plsc.subcore_barrier()
    pltpu.sync_copy(acc.at[pl.ds(s * _RPT, _RPT)],
                    out_sum.at[pl.ds(c * _NPAD + s * _RPT, _RPT)])
    pltpu.sync_copy(cacc.at[pl.ds(s * _RPT, _RPT)],
                    out_cnt.at[pl.ds(c * _NPAD + s * _RPT, _RPT)])


_sweep0 = functools.partial(
    pl.kernel,
    _sweep0_body,
    out_type=[jax.ShapeDtypeStruct((_NC * _NPAD, _H), _f32),
              jax.ShapeDtypeStruct((_NC * _NPAD,), _f32)],
    mesh=_mesh(),
    scratch_types=[
        pltpu.VMEM_SHARED((_NPAD, _H), _f32),
        pltpu.VMEM_SHARED((_NPAD,), _f32),
        pltpu.VMEM((_CHS, _H), _f32),
        pltpu.VMEM((_CHS, _H), _f32),
        pltpu.VMEM((_CHS,), jnp.int32),
        pltpu.VMEM((_CHS,), jnp.int32),
        pltpu.VMEM((_EPT,), jnp.int32),
        pltpu.VMEM((_CHS,), _f32),
        pltpu.SemaphoreType.DMA,
        pltpu.SemaphoreType.DMA,
        pltpu.SemaphoreType.DMA,
        pltpu.SemaphoreType.DMA,
    ],
)()


def _sweep1_body(tab, src, dst, zrows, selfh, rch, gidx_t, gidx_l,
                 gsum, gself, grc,
                 acc, rows0, rows1, dch0, dch1, src_all, idxb,
                 semg0, semg1, semd0, semd1):
    c = lax.axis_index("c")
    s = lax.axis_index("s")
    pltpu.sync_copy(zrows, acc.at[pl.ds(s * _RPT, _RPT)])
    base = c * _EP + s * _EPT
    plsc.subcore_barrier()
    _edge_sweep(tab, src, dst, base, src_all, dch0, dch1,
                acc, rows0, rows1, semg0, semg1, semd0, semd1,
                lambda dch: None)
    plsc.subcore_barrier()
    g0 = rows0.at[pl.ds(0, 64)]
    g1 = rows1.at[pl.ds(0, 64)]
    for t in range(_B // (_NS * 64)):
        off = c * _B + s * (_B // _NS) + t * 64
        pltpu.sync_copy(gidx_l.at[pl.ds(off, 64)], idxb)
        pltpu.async_copy(acc.at[idxb], g0, semg0).wait()
        pltpu.sync_copy(g0, gsum.at[pl.ds(off, 64)])
        pltpu.sync_copy(gidx_t.at[pl.ds(off, 64)], idxb)
        pltpu.async_copy(selfh.at[idxb], g0, semg0).wait()
        pltpu.sync_copy(g0, gself.at[pl.ds(off, 64)])
        pltpu.async_copy(rch.at[idxb], g1, semg1).wait()
        pltpu.sync_copy(g1, grc.at[pl.ds(off, 64)])


_sweep1 = functools.partial(
    pl.kernel,
    _sweep1_body,
    out_type=[jax.ShapeDtypeStruct((_NC * _B, _H), _f32)] * 3,
    mesh=_mesh(),
    scratch_types=[
        pltpu.VMEM_SHARED((_NPAD, _H), _f32),
        pltpu.VMEM((_CHS, _H), _f32),
        pltpu.VMEM((_CHS, _H), _f32),
        pltpu.VMEM((_CHS,), jnp.int32),
        pltpu.VMEM((_CHS,), jnp.int32),
        pltpu.VMEM((_EPT,), jnp.int32),
        pltpu.VMEM((64,), jnp.int32),
        pltpu.SemaphoreType.DMA,
        pltpu.SemaphoreType.DMA,
        pltpu.SemaphoreType.DMA,
        pltpu.SemaphoreType.DMA,
    ],
)()


def _d0_body(sums, cnt, xt, w, b, l1tab, selfo, rco):
    dot = functools.partial(jnp.dot, preferred_element_type=_f32)
    j = pl.program_id(0)
    rc = 1.0 / jnp.maximum(cnt[:, 0:1], 1.0)
    agg = sums[...] * rc
    e0 = jnp.maximum(dot(agg, w[0, 0]) + b[0, 0] + dot(xt[...], w[0, 1]), 0.0)
    e1 = jnp.maximum(dot(agg, w[0, 2]) + b[0, 1] + dot(xt[...], w[0, 3]), 0.0)
    is0 = (j == 0)
    l1tab[...] = jnp.where(is0, e0, e1)
    selfo[...] = jnp.where(is0, e1, e0)
    rco[...] = jnp.broadcast_to(rc, rco.shape)


def _run_d0(sums, cnt, xt, w, b):
    r = 1024
    nb = _NPAD // r
    side = pl.BlockSpec((r, _H), lambda j, i: (j * nb + i, 0))
    cblk = pl.BlockSpec((r, 1), lambda j, i: (j * nb + i, 0))
    oth = pl.BlockSpec((r, _H), lambda j, i: ((1 - j) * nb + i, 0))
    wblk = pl.BlockSpec((1, 4, _H, _H), lambda j, i: (j, 0, 0, 0))
    bblk = pl.BlockSpec((1, 2, _H), lambda j, i: (j, 0, 0))
    return pl.pallas_call(
        _d0_body,
        grid=(2, nb),
        in_specs=[side, cblk, oth, wblk, bblk],
        out_specs=[side, oth, oth],
        out_shape=[jax.ShapeDtypeStruct((_NC * _NPAD, _H), _f32)] * 3,
    )(sums, cnt, xt, w, b)


def _d1_body(gd0, gdx, gcd, gs0, gsx, gcs,
             w_d, b_d, w_s, b_s, wv_t, bv, wo_t, bo,
             w1, b1, w2, b2, w3r, out):
    dot = functools.partial(jnp.dot, preferred_element_type=_f32)
    demb = jnp.maximum(
        dot(gd0[...] * gcd[...], w_d[0]) + b_d[0] + dot(gdx[...], w_d[1]),
        0.0)
    semb = jnp.maximum(
        dot(gs0[...] * gcs[...], w_s[0]) + b_s[0] + dot(gsx[...], w_s[1]),
        0.0)
    datt = dot(dot(semb, wv_t[0]) + bv[0], wo_t[0]) + bo[0]
    satt = dot(dot(demb, wv_t[1]) + bv[1], wo_t[1]) + bo[1]
    li = jnp.concatenate([demb, semb, datt, satt], axis=-1)
    h = jnp.maximum(dot(li, w1[...]) + b1[0], 0.0)
    h = jnp.maximum(dot(h, w2[...]) + b2[0], 0.0)
    o = jnp.sum(h * w3r[0:1, :], axis=-1) + w3r[1, 0]
    out[...] = o.reshape(out.shape)


def _run_d1(gsum, gself, grc,
            w_d, b_d, w_s, b_s, wv_t, bv, wo_t, bo, w1, b1, w2, b2, w3r):
    r = 1024
    nb = _B // r
    grid = (nb,)
    drow = pl.BlockSpec((r, _H), lambda i: (i, 0))
    srow = pl.BlockSpec((r, _H), lambda i: (nb + i, 0))
    w2blk = pl.BlockSpec((2, _H, _H), lambda i: (0, 0, 0))
    bblk = pl.BlockSpec((2, _H), lambda i: (0, 0))
    full = lambda a: pl.BlockSpec(a.shape, lambda i: tuple(0 for _ in a.shape))
    return pl.pallas_call(
        _d1_body,
        grid=grid,
        in_specs=[drow, drow, drow, srow, srow, srow,
                  w2blk, bblk, w2blk, bblk, w2blk, bblk, w2blk, bblk,
                  full(w1), full(b1), full(w2), full(b2), full(w3r)],
        out_specs=[pl.BlockSpec((r // _H, _H), lambda i: (i, 0))],
        out_shape=[jax.ShapeDtypeStruct((_B // _H, _H), _f32)],
    )(gsum, gself, grc, gsum, gself, grc,
      w_d, b_d, w_s, b_s, wv_t, bv, wo_t, bo, w1, b1, w2, b2, w3r)[0]


def _pad_edges(src, dst, src_off):
    npd = _EP - _E
    i32 = jnp.int32
    srcp = jnp.concatenate(
        [src.astype(i32) + src_off, jnp.full((npd,), src_off, i32)])
    pad_dst = _N + (jnp.arange(npd, dtype=i32) % (_NPAD - _N))
    dstp = jnp.concatenate([dst.astype(i32), pad_dst])
    return srcp, dstp


def kernel(edge_index_dd, edge_index_rev, drug_idx, disease_idx, drug_table,
           disease_table, sage_wl, sage_bl, sage_wr, attn_in_w, attn_in_b,
           attn_out_w, attn_out_b, mlp_w1, mlp_b1, mlp_w2, mlp_b2, mlp_w3,
           mlp_b3):
    i32 = jnp.int32
    h = _H
    di = drug_idx.astype(i32)
    si = disease_idx.astype(i32)

    src_dd0, dst_dd = _pad_edges(edge_index_dd[0], edge_index_dd[1], 0)
    src_ddp, _ = _pad_edges(edge_index_dd[0], edge_index_dd[1], _NPAD)
    src_rv0, dst_rv = _pad_edges(edge_index_rev[0], edge_index_rev[1], 0)
    src_rvp, _ = _pad_edges(edge_index_rev[0], edge_index_rev[1], _NPAD)

    T = jnp.zeros((_NC * _NPAD, _H), _f32)
    T = T.at[:_N].set(drug_table).at[_NPAD:_NPAD + _N].set(disease_table)

    src0 = jnp.concatenate([src_dd0, src_rvp])
    dst0 = jnp.concatenate([dst_dd, dst_rv])
    zrows = jnp.zeros((_RPT, _H), _f32)
    zc = jnp.zeros((_RPT,), _f32)
    ones_h = jnp.ones((_CHS,), _f32)

    sums0, cnt0 = _sweep0(T, src0, dst0, zrows, zc, ones_h)

    w = jnp.stack([
        jnp.stack([sage_wl[0, 0, 0], sage_wr[0, 0, 0],
                   sage_wl[1, 0, 0], sage_wr[1, 0, 0]]),
        jnp.stack([sage_wl[0, 0, 1], sage_wr[0, 0, 1],
                   sage_wl[1, 0, 1], sage_wr[1, 0, 1]]),
    ])
    bb = jnp.stack([
        jnp.stack([sage_bl[0, 0, 0], sage_bl[1, 0, 0]]),
        jnp.stack([sage_bl[0, 0, 1], sage_bl[1, 0, 1]]),
    ])

    l1tab, selfh, rch = _run_d0(sums0, cnt0.reshape(-1, 1), T, w, bb)

    src1 = jnp.concatenate([src_rv0, src_ddp])
    dst1 = jnp.concatenate([dst_rv, dst_dd])
    gidx_t = jnp.concatenate([di, si + _NPAD])
    gidx_l = jnp.concatenate([di, si])

    gsum, gself, grc = _sweep1(l1tab, src1, dst1, zrows, selfh, rch,
                               gidx_t, gidx_l)

    w_d = jnp.stack([sage_wl[0, 1, 1], sage_wr[0, 1, 1]])
    b_d = jnp.stack([sage_bl[0, 1, 1], sage_bl[0, 1, 1]])
    w_s = jnp.stack([sage_wl[1, 1, 0], sage_wr[1, 1, 0]])
    b_s = jnp.stack([sage_bl[1, 1, 0], sage_bl[1, 1, 0]])
    wv_t = jnp.stack([attn_in_w[0, 2 * h:3 * h].T, attn_in_w[1, 2 * h:3 * h].T])
    bv = jnp.stack([attn_in_b[0, 2 * h:3 * h], attn_in_b[1, 2 * h:3 * h]])
    wo_t = jnp.stack([attn_out_w[0].T, attn_out_w[1].T])
    bo = jnp.stack([attn_out_b[0], attn_out_b[1]])
    b1 = mlp_b1.reshape(1, -1)
    b2 = mlp_b2.reshape(1, -1)
    w3r = jnp.concatenate(
        [mlp_w3[:, 0:1].T, jnp.full((1, _H), mlp_b3[0], _f32)], axis=0)

    out = _run_d1(gsum, gself, grc,
                  w_d, b_d, w_s, b_s, wv_t, bv, wo_t, bo,
                  mlp_w1, b1, mlp_w2, b2, w3r)
    return out.reshape(_B)


# speedup vs baseline: 2.0687x; 2.0687x over previous
import functools

import jax
import jax.numpy as jnp
from jax import lax
from jax.experimental import pallas as pl
from jax.experimental.pallas import tpu as pltpu
from jax.experimental.pallas import tpu_sc as plsc

_N = 10000
_NPAD = 10240
_E = 320000
_H = 128
_B = 4096
_NC = 2
_NS = 16
_CH = 80
_EPT = _E // (_NC * _NS)
_NCH = _EPT // _CH
_RPT = _NPAD // _NS
_GC = 64
_GPT = _B // _NS

_f32 = jnp.float32


def _mesh():
    return plsc.VectorSubcoreMesh(core_axis_name="c", subcore_axis_name="s")


def _edge_sweep(tab, src, dst, base, src_all, dch0, dch1,
                acc, rows0, rows1, semg0, semg1, semd0, semd1, scat_x):
    def fire(i, dch, rows, semd, semg):
        pltpu.async_copy(dst.at[pl.ds(base + i * _CH, _CH)], dch, semd)
        pltpu.async_copy(tab.at[src_all.at[pl.ds(i * _CH, _CH)]],
                         rows, semg)

    def drain(dch, rows, semd, semg):
        pltpu.make_async_copy(dst.at[pl.ds(base, _CH)], dch, semd).wait()
        pltpu.make_async_copy(tab.at[src_all.at[pl.ds(0, _CH)]],
                              rows, semg).wait()

    def scat(dch, rows):
        pltpu.sync_copy(rows, acc.at[dch], add=True)
        scat_x(dch)

    pltpu.sync_copy(src.at[pl.ds(base, _EPT)], src_all)
    fire(0, dch0, rows0, semd0, semg0)

    def step(k, carry):
        i0 = k * 2
        fire(i0 + 1, dch1, rows1, semd1, semg1)
        drain(dch0, rows0, semd0, semg0)
        scat(dch0, rows0)
        fire(i0 + 2, dch0, rows0, semd0, semg0)
        drain(dch1, rows1, semd1, semg1)
        scat(dch1, rows1)
        return carry

    lax.fori_loop(0, (_NCH - 1) // 2, step, 0)
    drain(dch0, rows0, semd0, semg0)
    scat(dch0, rows0)


def _seg0_body(tab, src, dst, zrows, zc, ones_h, out_sum, out_cnt,
               acc, cacc, rows0, rows1, dch0, dch1, src_all, ones_v,
               semg0, semg1, semd0, semd1):
    c = lax.axis_index("c")
    s = lax.axis_index("s")
    pltpu.sync_copy(zrows, acc.at[pl.ds(s * _RPT, _RPT)])
    pltpu.sync_copy(zc, cacc.at[pl.ds(s * _RPT, _RPT)])
    pltpu.sync_copy(ones_h, ones_v)
    base = c * (_E // _NC) + s * _EPT
    plsc.subcore_barrier()

    def scat_cnt(dch):
        pltpu.sync_copy(ones_v, cacc.at[dch], add=True)

    _edge_sweep(tab, src, dst, base, src_all, dch0, dch1,
                acc, rows0, rows1, semg0, semg1, semd0, semd1, scat_cnt)
    plsc.subcore_barrier()
    pltpu.sync_copy(acc.at[pl.ds(s * _RPT, _RPT)],
                    out_sum.at[pl.ds(c * _NPAD + s * _RPT, _RPT)])
    pltpu.sync_copy(cacc.at[pl.ds(s * _RPT, _RPT)],
                    out_cnt.at[pl.ds(c * _NPAD + s * _RPT, _RPT)])


_seg0 = functools.partial(
    pl.kernel,
    _seg0_body,
    out_type=[jax.ShapeDtypeStruct((_NC * _NPAD, _H), _f32),
              jax.ShapeDtypeStruct((_NC * _NPAD,), _f32)],
    mesh=_mesh(),
    scratch_types=[
        pltpu.VMEM_SHARED((_NPAD, _H), _f32),
        pltpu.VMEM_SHARED((_NPAD,), _f32),
        pltpu.VMEM((_CH, _H), _f32),
        pltpu.VMEM((_CH, _H), _f32),
        pltpu.VMEM((_CH,), jnp.int32),
        pltpu.VMEM((_CH,), jnp.int32),
        pltpu.VMEM((_EPT,), jnp.int32),
        pltpu.VMEM((_CH,), _f32),
        pltpu.SemaphoreType.DMA,
        pltpu.SemaphoreType.DMA,
        pltpu.SemaphoreType.DMA,
        pltpu.SemaphoreType.DMA,
    ],
)()


def _seg1_body(tab, src, dst, zrows, selfh, rch, gidx,
               gsum, gx, grc,
               acc, rows0, rows1, dch0, dch1, src_all, idxb,
               semg0, semg1, semd0, semd1):
    c = lax.axis_index("c")
    s = lax.axis_index("s")
    pltpu.sync_copy(zrows, acc.at[pl.ds(s * _RPT, _RPT)])
    base = c * (_E // _NC) + s * _EPT
    plsc.subcore_barrier()
    _edge_sweep(tab, src, dst, base, src_all, dch0, dch1,
                acc, rows0, rows1, semg0, semg1, semd0, semd1,
                lambda dch: None)
    plsc.subcore_barrier()
    g0 = rows0.at[pl.ds(0, _GC)]
    g1 = rows1.at[pl.ds(0, _GC)]
    for t in range(_GPT // _GC):
        off = s * _GPT + t * _GC
        pltpu.sync_copy(gidx.at[pl.ds(off, _GC)], idxb)
        pltpu.async_copy(acc.at[idxb], g0, semg0).wait()
        pltpu.sync_copy(g0, gsum.at[pl.ds(c * _B + off, _GC)])

        @pl.when(c == 0)
        def _():
            pltpu.async_copy(selfh.at[idxb], g1, semg1).wait()
            pltpu.sync_copy(g1, gx.at[pl.ds(off, _GC)])

        @pl.when(c == 1)
        def _():
            pltpu.async_copy(rch.at[idxb], g1, semg1).wait()
            pltpu.sync_copy(g1, grc.at[pl.ds(off, _GC)])


_seg1 = functools.partial(
    pl.kernel,
    _seg1_body,
    out_type=[jax.ShapeDtypeStruct((_NC * _B, _H), _f32),
              jax.ShapeDtypeStruct((_B, _H), _f32),
              jax.ShapeDtypeStruct((_B, _H), _f32)],
    mesh=_mesh(),
    scratch_types=[
        pltpu.VMEM_SHARED((_NPAD, _H), _f32),
        pltpu.VMEM((_CH, _H), _f32),
        pltpu.VMEM((_CH, _H), _f32),
        pltpu.VMEM((_CH,), jnp.int32),
        pltpu.VMEM((_CH,), jnp.int32),
        pltpu.VMEM((_EPT,), jnp.int32),
        pltpu.VMEM((_GC,), jnp.int32),
        pltpu.SemaphoreType.DMA,
        pltpu.SemaphoreType.DMA,
        pltpu.SemaphoreType.DMA,
        pltpu.SemaphoreType.DMA,
    ],
)()


def _d0_body(sa_dis, sb_dis, cda, cdb, xt_dis, sa_drug, sb_drug, cga, cgb,
             xt_drug, w_dis, b_dis, w_drug, b_drug,
             xs0, xs1, xd0, xd1, rc_dis, rc_drug):
    dot = functools.partial(jnp.dot, preferred_element_type=_f32)
    rcd = 1.0 / jnp.maximum(cda[:, 0:1] + cdb[:, 0:1], 1.0)
    rcg = 1.0 / jnp.maximum(cga[:, 0:1] + cgb[:, 0:1], 1.0)
    rc_dis[...] = jnp.broadcast_to(rcd, rc_dis.shape)
    rc_drug[...] = jnp.broadcast_to(rcg, rc_drug.shape)
    agg_dis = (sa_dis[...] + sb_dis[...]) * rcd
    agg_drug = (sa_drug[...] + sb_drug[...]) * rcg
    xs0[...] = jnp.maximum(
        dot(agg_dis, w_dis[0]) + b_dis[0] + dot(xt_dis[...], w_dis[1]), 0.0)
    xs1[...] = jnp.maximum(
        dot(agg_dis, w_dis[2]) + b_dis[1] + dot(xt_dis[...], w_dis[3]), 0.0)
    xd0[...] = jnp.maximum(
        dot(agg_drug, w_drug[0]) + b_drug[0] + dot(xt_drug[...], w_drug[1]), 0.0)
    xd1[...] = jnp.maximum(
        dot(agg_drug, w_drug[2]) + b_drug[1] + dot(xt_drug[...], w_drug[3]), 0.0)


def _run_d0(sa_dis, sb_dis, cda, cdb, xt_dis, sa_drug, sb_drug, cga, cgb,
            xt_drug, w_dis, b_dis, w_drug, b_drug):
    r = 1000
    grid = (_N // r,)
    row = pl.BlockSpec((r, _H), lambda i: (i, 0))
    cblk = pl.BlockSpec((r, 1), lambda i: (i, 0))
    wblk = pl.BlockSpec((4, _H, _H), lambda i: (0, 0, 0))
    bblk = pl.BlockSpec((2, _H), lambda i: (0, 0))
    return pl.pallas_call(
        _d0_body,
        grid=grid,
        in_specs=[row, row, cblk, cblk, row, row, row, cblk, cblk, row,
                  wblk, bblk, wblk, bblk],
        out_specs=[row, row, row, row, row, row],
        out_shape=[jax.ShapeDtypeStruct((_N, _H), _f32)] * 6,
    )(sa_dis, sb_dis, cda, cdb, xt_dis, sa_drug, sb_drug, cga, cgb,
      xt_drug, w_dis, b_dis, w_drug, b_drug)


def _d1_body(gd0, gd1, gdx, gcd, gs0, gs1, gsx, gcs,
             w_d, b_d, w_s, b_s, wv_t, bv, wo_t, bo,
             w1, b1, w2, b2, w3r, out):
    dot = functools.partial(jnp.dot, preferred_element_type=_f32)
    demb = jnp.maximum(
        dot((gd0[...] + gd1[...]) * gcd[...], w_d[0])
        + b_d[0] + dot(gdx[...], w_d[1]), 0.0)
    semb = jnp.maximum(
        dot((gs0[...] + gs1[...]) * gcs[...], w_s[0])
        + b_s[0] + dot(gsx[...], w_s[1]), 0.0)
    datt = dot(dot(semb, wv_t[0]) + bv[0], wo_t[0]) + bo[0]
    satt = dot(dot(demb, wv_t[1]) + bv[1], wo_t[1]) + bo[1]
    li = jnp.concatenate([demb, semb, datt, satt], axis=-1)
    h = jnp.maximum(dot(li, w1[...]) + b1[0], 0.0)
    h = jnp.maximum(dot(h, w2[...]) + b2[0], 0.0)
    o = jnp.sum(h * w3r[0:1, :], axis=-1) + w3r[1, 0]
    out[...] = o.reshape(out.shape)


def _run_d1(gsum_d, gx_d, grc_d, gsum_s, gx_s, grc_s,
            w_d, b_d, w_s, b_s, wv_t, bv, wo_t, bo, w1, b1, w2, b2, w3r):
    r = 1024
    nb = _B // r
    arow = pl.BlockSpec((r, _H), lambda i: (i, 0))
    brow = pl.BlockSpec((r, _H), lambda i: (nb + i, 0))
    w2blk = pl.BlockSpec((2, _H, _H), lambda i: (0, 0, 0))
    bblk = pl.BlockSpec((2, _H), lambda i: (0, 0))
    full = lambda a: pl.BlockSpec(a.shape, lambda i: tuple(0 for _ in a.shape))
    return pl.pallas_call(
        _d1_body,
        grid=(nb,),
        in_specs=[arow, brow, arow, arow, arow, brow, arow, arow,
                  w2blk, bblk, w2blk, bblk, w2blk, bblk, w2blk, bblk,
                  full(w1), full(b1), full(w2), full(b2), full(w3r)],
        out_specs=[pl.BlockSpec((r // _H, _H), lambda i: (i, 0))],
        out_shape=[jax.ShapeDtypeStruct((_B // _H, _H), _f32)],
    )(gsum_d, gsum_d, gx_d, grc_d, gsum_s, gsum_s, gx_s, grc_s,
      w_d, b_d, w_s, b_s, wv_t, bv, wo_t, bo, w1, b1, w2, b2, w3r)[0]


def kernel(edge_index_dd, edge_index_rev, drug_idx, disease_idx, drug_table,
           disease_table, sage_wl, sage_bl, sage_wr, attn_in_w, attn_in_b,
           attn_out_w, attn_out_b, mlp_w1, mlp_b1, mlp_w2, mlp_b2, mlp_w3,
           mlp_b3):
    i32 = jnp.int32
    h = _H
    src_dd = edge_index_dd[0].astype(i32)
    dst_dd = edge_index_dd[1].astype(i32)
    src_rv = edge_index_rev[0].astype(i32)
    dst_rv = edge_index_rev[1].astype(i32)
    di = drug_idx.astype(i32)
    si = disease_idx.astype(i32)

    zrows = jnp.zeros((_RPT, _H), _f32)
    zc = jnp.zeros((_RPT,), _f32)
    ones_h = jnp.ones((_CH,), _f32)

    sum_dis0, cnt_dd = _seg0(drug_table, src_dd, dst_dd, zrows, zc, ones_h)
    sum_drug0, cnt_rv = _seg0(disease_table, src_rv, dst_rv, zrows, zc, ones_h)

    w_dis = jnp.stack([sage_wl[0, 0, 0], sage_wr[0, 0, 0],
                       sage_wl[1, 0, 0], sage_wr[1, 0, 0]])
    b_dis = jnp.stack([sage_bl[0, 0, 0], sage_bl[1, 0, 0]])
    w_drug = jnp.stack([sage_wl[0, 0, 1], sage_wr[0, 0, 1],
                        sage_wl[1, 0, 1], sage_wr[1, 0, 1]])
    b_drug = jnp.stack([sage_bl[0, 0, 1], sage_bl[1, 0, 1]])

    xs0, xs1, xd0, xd1, rc_dis, rc_drug = _run_d0(
        sum_dis0[:_N], sum_dis0[_NPAD:_NPAD + _N],
        cnt_dd[:_N].reshape(_N, 1), cnt_dd[_NPAD:_NPAD + _N].reshape(_N, 1),
        disease_table,
        sum_drug0[:_N], sum_drug0[_NPAD:_NPAD + _N],
        cnt_rv[:_N].reshape(_N, 1), cnt_rv[_NPAD:_NPAD + _N].reshape(_N, 1),
        drug_table,
        w_dis, b_dis, w_drug, b_drug)

    gsum_d, gx_d, grc_d = _seg1(xs0, src_rv, dst_rv, zrows, xd0, rc_drug, di)
    gsum_s, gx_s, grc_s = _seg1(xd1, src_dd, dst_dd, zrows, xs1, rc_dis, si)

    w_d = jnp.stack([sage_wl[0, 1, 1], sage_wr[0, 1, 1]])
    b_d = jnp.stack([sage_bl[0, 1, 1], sage_bl[0, 1, 1]])
    w_s = jnp.stack([sage_wl[1, 1, 0], sage_wr[1, 1, 0]])
    b_s = jnp.stack([sage_bl[1, 1, 0], sage_bl[1, 1, 0]])
    wv_t = jnp.stack([attn_in_w[0, 2 * h:3 * h].T, attn_in_w[1, 2 * h:3 * h].T])
    bv = jnp.stack([attn_in_b[0, 2 * h:3 * h], attn_in_b[1, 2 * h:3 * h]])
    wo_t = jnp.stack([attn_out_w[0].T, attn_out_w[1].T])
    bo = jnp.stack([attn_out_b[0], attn_out_b[1]])
    b1 = mlp_b1.reshape(1, -1)
    b2 = mlp_b2.reshape(1, -1)
    w3r = jnp.concatenate(
        [mlp_w3[:, 0:1].T, jnp.full((1, _H), mlp_b3[0], _f32)], axis=0)

    out = _run_d1(gsum_d, gx_d, grc_d, gsum_s, gx_s, grc_s,
                  w_d, b_d, w_s, b_s, wv_t, bv, wo_t, bo,
                  mlp_w1, b1, mlp_w2, b2, w3r)
    return out.reshape(_B)


# speedup vs baseline: 2.4542x; 1.1864x over previous
import functools

import jax
import jax.numpy as jnp
from jax import lax
from jax.experimental import pallas as pl
from jax.experimental.pallas import tpu as pltpu
from jax.experimental.pallas import tpu_sc as plsc

_N = 10000
_NPAD = 10240
_E = 320000
_H = 128
_B = 4096
_NC = 2
_NS = 16
_CH = 80
_EPT = _E // (_NC * _NS)
_NCH = _EPT // _CH
_RPT = _NPAD // _NS
_GC = 64
_GPT = _B // _NS

_f32 = jnp.float32


def _mesh():
    return plsc.VectorSubcoreMesh(core_axis_name="c", subcore_axis_name="s")


def _edge_sweep(tab, src, dst, base, src_all, slots, acc, scat_x):
    def fire(i, b):
        dch, rows, semd, semg = slots[b]
        pltpu.async_copy(dst.at[pl.ds(base + i * _CH, _CH)], dch, semd)
        pltpu.async_copy(tab.at[src_all.at[pl.ds(i * _CH, _CH)]],
                         rows, semg)

    def drain_scat(b):
        dch, rows, semd, semg = slots[b]
        pltpu.make_async_copy(dst.at[pl.ds(base, _CH)], dch, semd).wait()
        pltpu.make_async_copy(tab.at[src_all.at[pl.ds(0, _CH)]],
                              rows, semg).wait()
        pltpu.sync_copy(rows, acc.at[dch], add=True)
        scat_x(dch)

    pltpu.sync_copy(src.at[pl.ds(base, _EPT)], src_all)
    fire(0, 0)
    fire(1, 1)

    def step(k, carry):
        i0 = k * 3
        for b in range(3):
            fire(i0 + b + 2, (b + 2) % 3)
            drain_scat(b)
        return carry

    lax.fori_loop(0, (_NCH - 2) // 3, step, 0)
    drain_scat(0)
    drain_scat(1)


def _seg0_body(tab, src, dst, zrows, zc, ones_h, out_sum, out_cnt,
               acc, cacc, rows0, rows1, rows2, dch0, dch1, dch2, src_all,
               ones_v, semg0, semg1, semg2, semd0, semd1, semd2):
    c = lax.axis_index("c")
    s = lax.axis_index("s")
    pltpu.sync_copy(zrows, acc.at[pl.ds(s * _RPT, _RPT)])
    pltpu.sync_copy(zc, cacc.at[pl.ds(s * _RPT, _RPT)])
    pltpu.sync_copy(ones_h, ones_v)
    base = c * (_E // _NC) + s * _EPT
    plsc.subcore_barrier()

    def scat_cnt(dch):
        pltpu.sync_copy(ones_v, cacc.at[dch], add=True)

    slots = [(dch0, rows0, semd0, semg0), (dch1, rows1, semd1, semg1),
             (dch2, rows2, semd2, semg2)]
    _edge_sweep(tab, src, dst, base, src_all, slots, acc, scat_cnt)
    plsc.subcore_barrier()
    pltpu.sync_copy(acc.at[pl.ds(s * _RPT, _RPT)],
                    out_sum.at[pl.ds(c * _NPAD + s * _RPT, _RPT)])
    pltpu.sync_copy(cacc.at[pl.ds(s * _RPT, _RPT)],
                    out_cnt.at[pl.ds(c * _NPAD + s * _RPT, _RPT)])


_seg0 = functools.partial(
    pl.kernel,
    _seg0_body,
    out_type=[jax.ShapeDtypeStruct((_NC * _NPAD, _H), _f32),
              jax.ShapeDtypeStruct((_NC * _NPAD,), _f32)],
    mesh=_mesh(),
    scratch_types=[
        pltpu.VMEM_SHARED((_NPAD, _H), _f32),
        pltpu.VMEM_SHARED((_NPAD,), _f32),
        pltpu.VMEM((_CH, _H), _f32),
        pltpu.VMEM((_CH, _H), _f32),
        pltpu.VMEM((_CH, _H), _f32),
        pltpu.VMEM((_CH,), jnp.int32),
        pltpu.VMEM((_CH,), jnp.int32),
        pltpu.VMEM((_CH,), jnp.int32),
        pltpu.VMEM((_EPT,), jnp.int32),
        pltpu.VMEM((_CH,), _f32),
    ] + [pltpu.SemaphoreType.DMA] * 6,
)()


def _seg1_body(tab, src, dst, zrows, selfh, rch, gidx,
               gsum, gx, grc,
               acc, rows0, rows1, rows2, dch0, dch1, dch2, src_all, idxb,
               semg0, semg1, semg2, semd0, semd1, semd2):
    c = lax.axis_index("c")
    s = lax.axis_index("s")
    pltpu.sync_copy(zrows, acc.at[pl.ds(s * _RPT, _RPT)])
    base = c * (_E // _NC) + s * _EPT
    plsc.subcore_barrier()
    slots = [(dch0, rows0, semd0, semg0), (dch1, rows1, semd1, semg1),
             (dch2, rows2, semd2, semg2)]
    _edge_sweep(tab, src, dst, base, src_all, slots, acc, lambda dch: None)
    plsc.subcore_barrier()
    g0 = rows0.at[pl.ds(0, _GC)]
    g1 = rows1.at[pl.ds(0, _GC)]
    for t in range(_GPT // _GC):
        off = s * _GPT + t * _GC
        pltpu.sync_copy(gidx.at[pl.ds(off, _GC)], idxb)
        pltpu.async_copy(acc.at[idxb], g0, semg0).wait()
        pltpu.sync_copy(g0, gsum.at[pl.ds(c * _B + off, _GC)])

        @pl.when(c == 0)
        def _():
            pltpu.async_copy(selfh.at[idxb], g1, semg1).wait()
            pltpu.sync_copy(g1, gx.at[pl.ds(off, _GC)])

        @pl.when(c == 1)
        def _():
            pltpu.async_copy(rch.at[idxb], g1, semg1).wait()
            pltpu.sync_copy(g1, grc.at[pl.ds(off, _GC)])


_seg1 = functools.partial(
    pl.kernel,
    _seg1_body,
    out_type=[jax.ShapeDtypeStruct((_NC * _B, _H), _f32),
              jax.ShapeDtypeStruct((_B, _H), _f32),
              jax.ShapeDtypeStruct((_B, _H), _f32)],
    mesh=_mesh(),
    scratch_types=[
        pltpu.VMEM_SHARED((_NPAD, _H), _f32),
        pltpu.VMEM((_CH, _H), _f32),
        pltpu.VMEM((_CH, _H), _f32),
        pltpu.VMEM((_CH, _H), _f32),
        pltpu.VMEM((_CH,), jnp.int32),
        pltpu.VMEM((_CH,), jnp.int32),
        pltpu.VMEM((_CH,), jnp.int32),
        pltpu.VMEM((_EPT,), jnp.int32),
        pltpu.VMEM((_GC,), jnp.int32),
    ] + [pltpu.SemaphoreType.DMA] * 6,
)()


def _d0_body(sa_dis, sb_dis, cda, cdb, xt_dis, sa_drug, sb_drug, cga, cgb,
             xt_drug, w_dis, b_dis, w_drug, b_drug,
             xs0, xs1, xd0, xd1, rc_dis, rc_drug):
    dot = functools.partial(jnp.dot, preferred_element_type=_f32)
    rcd = 1.0 / jnp.maximum(cda[:, 0:1] + cdb[:, 0:1], 1.0)
    rcg = 1.0 / jnp.maximum(cga[:, 0:1] + cgb[:, 0:1], 1.0)
    rc_dis[...] = jnp.broadcast_to(rcd, rc_dis.shape)
    rc_drug[...] = jnp.broadcast_to(rcg, rc_drug.shape)
    agg_dis = (sa_dis[...] + sb_dis[...]) * rcd
    agg_drug = (sa_drug[...] + sb_drug[...]) * rcg
    xs0[...] = jnp.maximum(
        dot(agg_dis, w_dis[0]) + b_dis[0] + dot(xt_dis[...], w_dis[1]), 0.0)
    xs1[...] = jnp.maximum(
        dot(agg_dis, w_dis[2]) + b_dis[1] + dot(xt_dis[...], w_dis[3]), 0.0)
    xd0[...] = jnp.maximum(
        dot(agg_drug, w_drug[0]) + b_drug[0] + dot(xt_drug[...], w_drug[1]), 0.0)
    xd1[...] = jnp.maximum(
        dot(agg_drug, w_drug[2]) + b_drug[1] + dot(xt_drug[...], w_drug[3]), 0.0)


def _run_d0(sa_dis, sb_dis, cda, cdb, xt_dis, sa_drug, sb_drug, cga, cgb,
            xt_drug, w_dis, b_dis, w_drug, b_drug):
    r = 1000
    grid = (_N // r,)
    row = pl.BlockSpec((r, _H), lambda i: (i, 0))
    cblk = pl.BlockSpec((r, 1), lambda i: (i, 0))
    wblk = pl.BlockSpec((4, _H, _H), lambda i: (0, 0, 0))
    bblk = pl.BlockSpec((2, _H), lambda i: (0, 0))
    return pl.pallas_call(
        _d0_body,
        grid=grid,
        in_specs=[row, row, cblk, cblk, row, row, row, cblk, cblk, row,
                  wblk, bblk, wblk, bblk],
        out_specs=[row, row, row, row, row, row],
        out_shape=[jax.ShapeDtypeStruct((_N, _H), _f32)] * 6,
    )(sa_dis, sb_dis, cda, cdb, xt_dis, sa_drug, sb_drug, cga, cgb,
      xt_drug, w_dis, b_dis, w_drug, b_drug)


def _d1_body(gd0, gd1, gdx, gcd, gs0, gs1, gsx, gcs,
             w_d, b_d, w_s, b_s, wv_t, bv, wo_t, bo,
             w1, b1, w2, b2, w3r, out):
    dot = functools.partial(jnp.dot, preferred_element_type=_f32)
    demb = jnp.maximum(
        dot((gd0[...] + gd1[...]) * gcd[...], w_d[0])
        + b_d[0] + dot(gdx[...], w_d[1]), 0.0)
    semb = jnp.maximum(
        dot((gs0[...] + gs1[...]) * gcs[...], w_s[0])
        + b_s[0] + dot(gsx[...], w_s[1]), 0.0)
    datt = dot(dot(semb, wv_t[0]) + bv[0], wo_t[0]) + bo[0]
    satt = dot(dot(demb, wv_t[1]) + bv[1], wo_t[1]) + bo[1]
    li = jnp.concatenate([demb, semb, datt, satt], axis=-1)
    h = jnp.maximum(dot(li, w1[...]) + b1[0], 0.0)
    h = jnp.maximum(dot(h, w2[...]) + b2[0], 0.0)
    o = jnp.sum(h * w3r[0:1, :], axis=-1) + w3r[1, 0]
    out[...] = o.reshape(out.shape)


def _run_d1(gsum_d, gx_d, grc_d, gsum_s, gx_s, grc_s,
            w_d, b_d, w_s, b_s, wv_t, bv, wo_t, bo, w1, b1, w2, b2, w3r):
    r = 1024
    nb = _B // r
    arow = pl.BlockSpec((r, _H), lambda i: (i, 0))
    brow = pl.BlockSpec((r, _H), lambda i: (nb + i, 0))
    w2blk = pl.BlockSpec((2, _H, _H), lambda i: (0, 0, 0))
    bblk = pl.BlockSpec((2, _H), lambda i: (0, 0))
    full = lambda a: pl.BlockSpec(a.shape, lambda i: tuple(0 for _ in a.shape))
    return pl.pallas_call(
        _d1_body,
        grid=(nb,),
        in_specs=[arow, brow, arow, arow, arow, brow, arow, arow,
                  w2blk, bblk, w2blk, bblk, w2blk, bblk, w2blk, bblk,
                  full(w1), full(b1), full(w2), full(b2), full(w3r)],
        out_specs=[pl.BlockSpec((r // _H, _H), lambda i: (i, 0))],
        out_shape=[jax.ShapeDtypeStruct((_B // _H, _H), _f32)],
    )(gsum_d, gsum_d, gx_d, grc_d, gsum_s, gsum_s, gx_s, grc_s,
      w_d, b_d, w_s, b_s, wv_t, bv, wo_t, bo, w1, b1, w2, b2, w3r)[0]


def kernel(edge_index_dd, edge_index_rev, drug_idx, disease_idx, drug_table,
           disease_table, sage_wl, sage_bl, sage_wr, attn_in_w, attn_in_b,
           attn_out_w, attn_out_b, mlp_w1, mlp_b1, mlp_w2, mlp_b2, mlp_w3,
           mlp_b3):
    i32 = jnp.int32
    h = _H
    src_dd = edge_index_dd[0].astype(i32)
    dst_dd = edge_index_dd[1].astype(i32)
    src_rv = edge_index_rev[0].astype(i32)
    dst_rv = edge_index_rev[1].astype(i32)
    di = drug_idx.astype(i32)
    si = disease_idx.astype(i32)

    zrows = jnp.zeros((_RPT, _H), _f32)
    zc = jnp.zeros((_RPT,), _f32)
    ones_h = jnp.ones((_CH,), _f32)

    sum_dis0, cnt_dd = _seg0(drug_table, src_dd, dst_dd, zrows, zc, ones_h)
    sum_drug0, cnt_rv = _seg0(disease_table, src_rv, dst_rv, zrows, zc, ones_h)

    w_dis = jnp.stack([sage_wl[0, 0, 0], sage_wr[0, 0, 0],
                       sage_wl[1, 0, 0], sage_wr[1, 0, 0]])
    b_dis = jnp.stack([sage_bl[0, 0, 0], sage_bl[1, 0, 0]])
    w_drug = jnp.stack([sage_wl[0, 0, 1], sage_wr[0, 0, 1],
                        sage_wl[1, 0, 1], sage_wr[1, 0, 1]])
    b_drug = jnp.stack([sage_bl[0, 0, 1], sage_bl[1, 0, 1]])

    xs0, xs1, xd0, xd1, rc_dis, rc_drug = _run_d0(
        sum_dis0[:_N], sum_dis0[_NPAD:_NPAD + _N],
        cnt_dd[:_N].reshape(_N, 1), cnt_dd[_NPAD:_NPAD + _N].reshape(_N, 1),
        disease_table,
        sum_drug0[:_N], sum_drug0[_NPAD:_NPAD + _N],
        cnt_rv[:_N].reshape(_N, 1), cnt_rv[_NPAD:_NPAD + _N].reshape(_N, 1),
        drug_table,
        w_dis, b_dis, w_drug, b_drug)

    gsum_d, gx_d, grc_d = _seg1(xs0, src_rv, dst_rv, zrows, xd0, rc_drug, di)
    gsum_s, gx_s, grc_s = _seg1(xd1, src_dd, dst_dd, zrows, xs1, rc_dis, si)

    w_d = jnp.stack([sage_wl[0, 1, 1], sage_wr[0, 1, 1]])
    b_d = jnp.stack([sage_bl[0, 1, 1], sage_bl[0, 1, 1]])
    w_s = jnp.stack([sage_wl[1, 1, 0], sage_wr[1, 1, 0]])
    b_s = jnp.stack([sage_bl[1, 1, 0], sage_bl[1, 1, 0]])
    wv_t = jnp.stack([attn_in_w[0, 2 * h:3 * h].T, attn_in_w[1, 2 * h:3 * h].T])
    bv = jnp.stack([attn_in_b[0, 2 * h:3 * h], attn_in_b[1, 2 * h:3 * h]])
    wo_t = jnp.stack([attn_out_w[0].T, attn_out_w[1].T])
    bo = jnp.stack([attn_out_b[0], attn_out_b[1]])
    b1 = mlp_b1.reshape(1, -1)
    b2 = mlp_b2.reshape(1, -1)
    w3r = jnp.concatenate(
        [mlp_w3[:, 0:1].T, jnp.full((1, _H), mlp_b3[0], _f32)], axis=0)

    out = _run_d1(gsum_d, gx_d, grc_d, gsum_s, gx_s, grc_s,
                  w_d, b_d, w_s, b_s, wv_t, bv, wo_t, bo,
                  mlp_w1, b1, mlp_w2, b2, w3r)
    return out.reshape(_B)
